# baseline (speedup 1.0000x reference)
import functools

import jax
import jax.numpy as jnp
from jax.experimental import pallas as pl
from jax.experimental.pallas import tpu as pltpu

_LAMBDA = 5.0
_Q = 16


def _monotone_key(x):
    i = jax.lax.bitcast_convert_type(x, jnp.int32)
    return i ^ ((i >> 31) & jnp.int32(0x7FFFFFFF))


def _topk_sum_rows2(xa, xb, k):
    ka = _monotone_key(xa)
    kb = _monotone_key(xb)
    R = xa.shape[0]
    sign = jnp.int32(-(2**31))

    ones = jnp.ones((xa.shape[1], 1), jnp.bfloat16)
    kf = jnp.float32(k)

    def body(i, carry):
        oa, ob = carry
        bit = jnp.int32(1) << (31 - i)
        ca = (oa | bit) ^ sign
        cb = (ob | bit) ^ sign
        cnt_a = jax.lax.dot_general(
            (ka >= ca).astype(jnp.bfloat16), ones, (((1,), (0,)), ((), ())),
            preferred_element_type=jnp.float32)
        cnt_b = jax.lax.dot_general(
            (kb >= cb).astype(jnp.bfloat16), ones, (((1,), (0,)), ((), ())),
            preferred_element_type=jnp.float32)
        return (jnp.where(cnt_a >= kf, oa | bit, oa),
                jnp.where(cnt_b >= kf, ob | bit, ob))

    zero = jnp.zeros((R, 1), jnp.int32)
    oa, ob = jax.lax.fori_loop(0, 32, body, (zero, zero))

    def finish(skey, obits):
        t_s = obits ^ sign
        gt = skey > t_s
        cnt_gt = jnp.sum(gt.astype(jnp.int32), axis=1, keepdims=True)
        x = jax.lax.bitcast_convert_type(
            skey ^ ((skey >> 31) & jnp.int32(0x7FFFFFFF)), jnp.float32)
        sum_gt = jnp.sum(jnp.where(gt, x, 0.0), axis=1, keepdims=True)
        tbits = t_s ^ ((t_s >> 31) & jnp.int32(0x7FFFFFFF))
        tval = jax.lax.bitcast_convert_type(tbits, jnp.float32)
        return sum_gt + (k - cnt_gt).astype(jnp.float32) * tval

    return finish(ka, oa), finish(kb, ob)


def _main_body(k, hl_ref, hlc_ref, tks_hl_ref, tks_hlc_ref, dp_ref):
    hl = hl_ref[...]
    hlc = hlc_ref[...]
    s_hl = jax.nn.sigmoid(hl)
    log_sig_hlc = jnp.minimum(hlc, 0.0) - jnp.log1p(jnp.exp(-jnp.abs(hlc)))
    dp_ref[...] = jnp.sum(s_hl * log_sig_hlc, axis=1, keepdims=True)
    tks_hl, tks_hlc = _topk_sum_rows2(hl, hlc, k)
    tks_hl_ref[...] = tks_hl
    tks_hlc_ref[...] = tks_hlc


def _finish_body(k, b, tks_hl_ref, tks_hlc_ref, dp_ref, y_ref,
                 total_ref, distill_ref, mil_hl_ref, mil_hlc_ref):
    y = y_ref[...]

    def bce_mean(x):
        return jnp.mean(jnp.maximum(x, 0.0) - x * y
                        + jnp.log1p(jnp.exp(-jnp.abs(x))))

    mil_hl = bce_mean(tks_hl_ref[...] * (1.0 / k))
    mil_hlc = bce_mean(tks_hlc_ref[...] * (1.0 / k))
    distill = -jnp.sum(dp_ref[...]) * (1.0 / b)
    total_ref[0] = _LAMBDA * distill + mil_hlc + mil_hl
    distill_ref[0] = distill
    mil_hl_ref[0] = mil_hl
    mil_hlc_ref[0] = mil_hlc


def kernel(logits_hl, logits_hlc, bag_labels):
    B, T, _ = logits_hl.shape
    k = max(T // _Q, 1)
    rows = 64
    grid = B // rows
    hl = logits_hl.reshape(B, T)
    hlc = logits_hlc.reshape(B, T)

    tks_hl, tks_hlc, dp = pl.pallas_call(
        functools.partial(_main_body, k),
        grid=(grid,),
        in_specs=[
            pl.BlockSpec((rows, T), lambda i: (i, 0)),
            pl.BlockSpec((rows, T), lambda i: (i, 0)),
        ],
        out_specs=[
            pl.BlockSpec((rows, 1), lambda i: (i, 0)),
            pl.BlockSpec((rows, 1), lambda i: (i, 0)),
            pl.BlockSpec((rows, 1), lambda i: (i, 0)),
        ],
        out_shape=[
            jax.ShapeDtypeStruct((B, 1), jnp.float32),
            jax.ShapeDtypeStruct((B, 1), jnp.float32),
            jax.ShapeDtypeStruct((B, 1), jnp.float32),
        ],
    )(hl, hlc)

    y = bag_labels.astype(jnp.float32).reshape(1, B)
    total, distill, mil_hl, mil_hlc = pl.pallas_call(
        functools.partial(_finish_body, k, B),
        in_specs=[
            pl.BlockSpec((1, B), lambda: (0, 0)),
            pl.BlockSpec((1, B), lambda: (0, 0)),
            pl.BlockSpec((1, B), lambda: (0, 0)),
            pl.BlockSpec((1, B), lambda: (0, 0)),
        ],
        out_specs=[
            pl.BlockSpec(memory_space=pltpu.SMEM),
            pl.BlockSpec(memory_space=pltpu.SMEM),
            pl.BlockSpec(memory_space=pltpu.SMEM),
            pl.BlockSpec(memory_space=pltpu.SMEM),
        ],
        out_shape=[jax.ShapeDtypeStruct((1,), jnp.float32)] * 4,
    )(tks_hl.reshape(1, B), tks_hlc.reshape(1, B), dp.reshape(1, B), y)

    return (total.reshape(()), distill.reshape(()),
            mil_hl.reshape(()), mil_hlc.reshape(()))
